# baseline (speedup 1.0000x reference)
import functools

import jax
import jax.numpy as jnp
from jax import lax
from jax.experimental import pallas as pl
from jax.experimental.pallas import tpu as pltpu

N_SPECIAL = 25
N_SEQ = 20
N_STRUCT = 20
NREG = N_SEQ * N_STRUCT
KTAB = 512
TB = 2048


def _body(ids_ref, shp_ref, f_ref, w1_ref, vecs_ref, out_ref, gcol_ref):
    blk = pl.program_id(0)
    df = lax.Precision.DEFAULT

    @pl.when(blk == 0)
    def _gate_table():
        F = f_ref[...]
        W1 = w1_ref[...]
        lnw = vecs_ref[0:1, :]
        lnb = vecs_ref[1:2, :]
        b1 = vecs_ref[2:3, :]
        w2 = vecs_ref[3:4, :]
        b2 = vecs_ref[4:5, 0:1]
        mu = jnp.mean(F, axis=1, keepdims=True)
        xc = F - mu
        var = jnp.mean(xc * xc, axis=1, keepdims=True)
        h0 = xc * lax.rsqrt(var + 1e-5) * lnw + lnb
        h1 = lax.dot(h0, W1, precision=df) + b1
        h1 = 0.5 * h1 * (1.0 + lax.erf(h1 * 0.7071067811865476))
        gate = jax.nn.sigmoid(jnp.sum(h1 * w2, axis=1, keepdims=True) + b2)
        gcol_ref[...] = jnp.broadcast_to(gate, (KTAB, 128))

    ids = ids_ref[...]
    shp = shp_ref[...]

    special = ids < N_SPECIAL
    ids_reg = jnp.clip(ids - N_SPECIAL, 0, NREG - 1)
    tgt = jnp.where(special, NREG + ids, ids_reg)
    seq = ids_reg // N_STRUCT

    jcol = lax.broadcasted_iota(jnp.int32, (TB, KTAB), 1)
    osel = (jcol == tgt).astype(jnp.float32)
    selmask = (jcol // N_STRUCT == seq).astype(jnp.float32)

    trow = lax.broadcasted_iota(jnp.int32, (N_STRUCT, KTAB), 0)
    tcol = lax.broadcasted_iota(jnp.int32, (N_STRUCT, KTAB), 1)
    T = (tcol % N_STRUCT == trow).astype(jnp.float32)
    shp_tiled = lax.dot(shp, T, precision=df)
    wsel = shp_tiled * selmask

    gate_t = lax.dot(osel, gcol_ref[...], precision=df)[:, 0:1]

    rows = lax.broadcasted_iota(jnp.int32, (TB, 1), 0)
    L = 2048
    pos = lax.rem(blk * TB + rows, L)
    regw = jnp.logical_and(pos >= 1, pos <= L - 2)
    keep = jnp.logical_or(special, regw)
    a = jnp.where(special, 1.0, jnp.where(regw, 1.0 - gate_t, 0.0))
    b = jnp.where(jnp.logical_and(jnp.logical_not(special), regw), gate_t, 0.0)
    selc = osel * a + wsel * b
    del keep
    out_ref[...] = lax.dot(selc, f_ref[...], precision=df)


@functools.partial(jax.jit, static_argnames=())
def kernel(input_ids, shp_tensor, special_embedding, full_embed, ln_w, ln_b, W1, b1, W2, b2):
    B, L = input_ids.shape
    D = special_embedding.shape[1]
    S = full_embed.shape[1]
    n_tok = B * L

    ids2 = input_ids.reshape(n_tok, 1)
    shp_full = jnp.zeros((B, L, S), dtype=shp_tensor.dtype)
    shp_full = shp_full.at[:, 1:L - 1, :].set(shp_tensor)
    shp2 = shp_full.reshape(n_tok, S)
    F = jnp.zeros((KTAB, D), dtype=jnp.float32)
    F = F.at[:NREG].set(full_embed.reshape(NREG, D))
    F = F.at[NREG:NREG + special_embedding.shape[0]].set(special_embedding)
    vecs = jnp.zeros((8, D), dtype=jnp.float32)
    vecs = vecs.at[0].set(ln_w).at[1].set(ln_b).at[2].set(b1)
    vecs = vecs.at[3].set(W2[:, 0]).at[4, 0].set(b2[0])

    grid = n_tok // TB
    out = pl.pallas_call(
        _body,
        grid=(grid,),
        in_specs=[
            pl.BlockSpec((TB, 1), lambda i: (i, 0)),
            pl.BlockSpec((TB, S), lambda i: (i, 0)),
            pl.BlockSpec((KTAB, D), lambda i: (0, 0)),
            pl.BlockSpec((D, D), lambda i: (0, 0)),
            pl.BlockSpec((8, D), lambda i: (0, 0)),
        ],
        out_specs=pl.BlockSpec((TB, D), lambda i: (i, 0)),
        out_shape=jax.ShapeDtypeStruct((n_tok, D), jnp.float32),
        scratch_shapes=[pltpu.VMEM((KTAB, 128), jnp.float32)],
    )(ids2, shp2, F, W1, vecs)
    return out.reshape(B, L, D)


# speedup vs baseline: 1.1297x; 1.1297x over previous
import functools

import jax
import jax.numpy as jnp
from jax import lax
from jax.experimental import pallas as pl
from jax.experimental.pallas import tpu as pltpu

N_SPECIAL = 25
N_SEQ = 20
N_STRUCT = 20
NREG = N_SEQ * N_STRUCT
NSP = 26
KTAB = 512
TB = 1024


def _body(ids_ref, shp_ref, f400_ref, spec_ref, w1_ref, lnw_ref, lnb_ref,
          b1_ref, w2_ref, b2_ref, out_ref, tab_ref, gcol_ref):
    blk = pl.program_id(0)
    df = lax.Precision.DEFAULT

    @pl.when(blk == 0)
    def _build_tables():
        tab_ref[0:NREG, :] = f400_ref[...]
        tab_ref[NREG:NREG + NSP, :] = spec_ref[...]
        tab_ref[NREG + NSP:, :] = jnp.zeros((KTAB - NREG - NSP, tab_ref.shape[1]),
                                            jnp.float32)
        F = tab_ref[...]
        mu = jnp.mean(F, axis=1, keepdims=True)
        xc = F - mu
        var = jnp.mean(xc * xc, axis=1, keepdims=True)
        h0 = xc * lax.rsqrt(var + 1e-5) * lnw_ref[...] + lnb_ref[...]
        h1 = lax.dot(h0, w1_ref[...], precision=df) + b1_ref[...]
        h1 = 0.5 * h1 * (1.0 + lax.erf(h1 * 0.7071067811865476))
        gate = jax.nn.sigmoid(lax.dot(h1, w2_ref[...], precision=df)
                              + b2_ref[...])
        gcol_ref[...] = jnp.broadcast_to(gate, (KTAB, 128))

    ids = ids_ref[...]
    shp = shp_ref[...]

    special = ids < N_SPECIAL
    ids_reg = jnp.clip(ids - N_SPECIAL, 0, NREG - 1)
    tgt = jnp.where(special, NREG + ids, ids_reg)
    seq = ids_reg // N_STRUCT

    jcol = lax.broadcasted_iota(jnp.int32, (TB, KTAB), 1)
    osel = (jcol == tgt).astype(jnp.float32)
    selmask = (jcol // N_STRUCT == seq).astype(jnp.float32)

    trow = lax.broadcasted_iota(jnp.int32, (N_STRUCT, KTAB), 0)
    tcol = lax.broadcasted_iota(jnp.int32, (N_STRUCT, KTAB), 1)
    T = (tcol % N_STRUCT == trow).astype(jnp.float32)
    shp_tiled = lax.dot(shp, T, precision=df)

    gate_t = lax.dot(osel, gcol_ref[...], precision=df)[:, 0:1]

    rows = lax.broadcasted_iota(jnp.int32, (TB, 1), 0)
    L = 2048
    pos = lax.rem(blk * TB + rows, L)
    regw = jnp.logical_and(pos >= 1, pos <= L - 2)
    a = jnp.where(special, 1.0, jnp.where(regw, 1.0 - gate_t, 0.0))
    b = jnp.where(jnp.logical_and(jnp.logical_not(special), regw), gate_t, 0.0)
    selc = osel * a + (shp_tiled * selmask) * b
    out_ref[...] = lax.dot(selc, tab_ref[...], precision=df)


@functools.partial(jax.jit, static_argnames=())
def kernel(input_ids, shp_tensor, special_embedding, full_embed, ln_w, ln_b, W1, b1, W2, b2):
    B, L = input_ids.shape
    D = special_embedding.shape[1]
    S = full_embed.shape[1]
    n_tok = B * L

    ids2 = input_ids.reshape(n_tok, 1)
    shp_full = jnp.zeros((B, L, S), dtype=shp_tensor.dtype)
    shp_full = shp_full.at[:, 1:L - 1, :].set(shp_tensor)
    shp2 = shp_full.reshape(n_tok, S)
    f400 = full_embed.reshape(NREG, D)

    grid = n_tok // TB
    const = lambda i: (0, 0)
    out = pl.pallas_call(
        _body,
        grid=(grid,),
        in_specs=[
            pl.BlockSpec((TB, 1), lambda i: (i, 0)),
            pl.BlockSpec((TB, S), lambda i: (i, 0)),
            pl.BlockSpec((NREG, D), const),
            pl.BlockSpec((NSP, D), const),
            pl.BlockSpec((D, D), const),
            pl.BlockSpec((1, D), const),
            pl.BlockSpec((1, D), const),
            pl.BlockSpec((1, D), const),
            pl.BlockSpec((D, 1), const),
            pl.BlockSpec((1, 1), const),
        ],
        out_specs=pl.BlockSpec((TB, D), lambda i: (i, 0)),
        out_shape=jax.ShapeDtypeStruct((n_tok, D), jnp.float32),
        scratch_shapes=[pltpu.VMEM((KTAB, D), jnp.float32),
                        pltpu.VMEM((KTAB, 128), jnp.float32)],
    )(ids2, shp2, f400, special_embedding, W1,
      ln_w.reshape(1, D), ln_b.reshape(1, D), b1.reshape(1, D),
      W2, b2.reshape(1, 1))
    return out.reshape(B, L, D)
